# baseline (speedup 1.0000x reference)
import functools

import jax
import jax.numpy as jnp
from jax import lax
from jax.experimental import pallas as pl
from jax.experimental.pallas import tpu as pltpu
from jax.experimental.pallas import tpu_sc as plsc

NC = 2
NS = 16
L = 16

NW = NC * NS


def _matmul_kernel(x_ref, w_ref, o_ref):
    o_ref[...] = jnp.dot(x_ref[...], w_ref[...],
                         preferred_element_type=jnp.float32)


def _combine_kernel(p_ref, b_ref, o_ref):
    o_ref[...] = p_ref[0] + p_ref[1] + b_ref[...]


def _lane_bcast(v16, j):
    idx = jnp.full((L, 1), j, jnp.int32)
    return lax.gather(
        v16, idx,
        lax.GatherDimensionNumbers(offset_dims=(), collapsed_slice_dims=(0,),
                                   start_index_map=(0,)),
        slice_sizes=(1,),
        mode=lax.GatherScatterMode.PROMISE_IN_BOUNDS)


def _make_sc_scatter(n, d, n_sup, sup, B):
    mesh = plsc.VectorSubcoreMesh(core_axis_name="c", subcore_axis_name="s")
    rc = B
    nrc = pl.cdiv(n, rc)
    rc_per_tile = pl.cdiv(nrc, NS)
    groups = B // L

    assert sup % 2 == 1

    @functools.partial(
        pl.kernel,
        out_type=jax.ShapeDtypeStruct((NC, n, d), jnp.float32),
        mesh=mesh,
        scratch_types=[
            pltpu.VMEM((sup, B), jnp.int32),
            pltpu.VMEM((sup * B,), jnp.int32),
            pltpu.VMEM((sup * B,), jnp.float32),
            pltpu.VMEM((B, d), jnp.float32),
            pltpu.VMEM((B, d), jnp.float32),
            pltpu.VMEM_SHARED((n, d), jnp.float32),
            pltpu.SemaphoreType.DMA,
            pltpu.SemaphoreType.DMA,
            pltpu.SemaphoreType.DMA,
            pltpu.SemaphoreType.DMA,
            pltpu.SemaphoreType.DMA,
            pltpu.SemaphoreType.DMA,
        ],
    )
    def sc_scatter(dst_hbm, ei_hbm, adj_hbm, hid_hbm, out_hbm,
                   dst_v, src_v, adj_v, rows0, rows1, acc,
                   g0a, g0b, g1a, g1b, s0, s1):
        c = lax.axis_index("c")
        s = lax.axis_index("s")
        wid = s * NC + c

        def _scale(rows, k):
            def _group(g, carry2):
                a16 = adj_v[pl.ds(k * B + g * L, L)]
                for jj in range(L):
                    av = _lane_bcast(a16, jj)
                    i = g * L + jj
                    for jd in range(d // L):
                        rows[i, pl.ds(jd * L, L)] = (
                            rows[i, pl.ds(jd * L, L)] * av)
                return carry2
            lax.fori_loop(0, groups, _group, 0)

        def _zrow(i, carry):
            for j in range(d // L):
                rows0[i, pl.ds(j * L, L)] = jnp.zeros((L,), jnp.float32)
            return carry
        lax.fori_loop(0, rc, _zrow, 0)
        for k in range(rc_per_tile):
            cid = k * NS + s

            @pl.when(cid < nrc)
            def _():
                off = pl.multiple_of(cid * rc, 8)
                pltpu.sync_copy(rows0, acc.at[pl.ds(off, rc)])
        plsc.subcore_barrier()

        h = B // 2

        def _g2(k, rows, sa, sb):
            pltpu.async_copy(hid_hbm.at[src_v.at[pl.ds(k * B, h)]],
                             rows.at[pl.ds(0, h)], sa)
            pltpu.async_copy(hid_hbm.at[src_v.at[pl.ds(k * B + h, h)]],
                             rows.at[pl.ds(h, h)], sb)

        def _w2(k, rows, sa, sb):
            pltpu.make_async_copy(hid_hbm.at[src_v.at[pl.ds(k * B, h)]],
                                  rows.at[pl.ds(0, h)], sa).wait()
            pltpu.make_async_copy(hid_hbm.at[src_v.at[pl.ds(k * B + h, h)]],
                                  rows.at[pl.ds(h, h)], sb).wait()

        def _sup(m, carry):
            eoff = wid * (n_sup * sup * B) + m * (sup * B)
            pltpu.sync_copy(dst_hbm.at[wid, m], dst_v)
            pltpu.sync_copy(ei_hbm.at[pl.ds(eoff, sup * B)], src_v)
            pltpu.sync_copy(adj_hbm.at[pl.ds(eoff, sup * B)], adj_v)
            _g2(0, rows0, g0a, g0b)

            def _pair(j, carry1):
                k0 = 2 * j
                k1 = 2 * j + 1
                _w2(k0, rows0, g0a, g0b)
                _g2(k1, rows1, g1a, g1b)
                _scale(rows0, k0)
                cp_s0 = pltpu.async_copy(rows0, acc.at[dst_v.at[k0]], s0,
                                         add=True)
                _w2(k1, rows1, g1a, g1b)
                cp_s0.wait()
                _g2(k1 + 1, rows0, g0a, g0b)
                _scale(rows1, k1)
                pltpu.async_copy(rows1, acc.at[dst_v.at[k1]], s1,
                                 add=True).wait()
                return carry1
            lax.fori_loop(0, sup // 2, _pair, 0)

            kt = sup - 1
            _w2(kt, rows0, g0a, g0b)
            _scale(rows0, kt)
            pltpu.sync_copy(rows0, acc.at[dst_v.at[kt]], add=True)
            return carry
        lax.fori_loop(0, n_sup, _sup, 0)
        plsc.subcore_barrier()

        for k in range(rc_per_tile):
            cid = k * NS + s

            @pl.when(cid < nrc)
            def _():
                off = pl.multiple_of(cid * rc, 8)
                pltpu.sync_copy(acc.at[pl.ds(off, rc)],
                                out_hbm.at[c, pl.ds(off, rc)])

    return sc_scatter


def kernel(x, edge_index, adj_values, W, b):
    n, d_in = x.shape
    d = W.shape[1]
    e = edge_index.shape[1]

    epw = e // NW
    B = 80
    sup = 25
    n_sup = epw // (B * sup)

    hidden = pl.pallas_call(
        _matmul_kernel,
        out_shape=jax.ShapeDtypeStruct((n, d), jnp.float32),
    )(x, W)

    dst_r = edge_index[0].reshape(NW, n_sup, sup, B)

    psum = _make_sc_scatter(n, d, n_sup, sup, B)(
        dst_r, edge_index[1], adj_values, hidden)

    out = pl.pallas_call(
        _combine_kernel,
        out_shape=jax.ShapeDtypeStruct((n, d), jnp.float32),
    )(psum, b.reshape(1, d))
    return out


# speedup vs baseline: 1.0403x; 1.0403x over previous
import functools

import jax
import jax.numpy as jnp
from jax import lax
from jax.experimental import pallas as pl
from jax.experimental.pallas import tpu as pltpu
from jax.experimental.pallas import tpu_sc as plsc

NC = 2
NS = 16
L = 16

NW = NC * NS


def _matmul_kernel(x_ref, w_ref, o_ref):
    o_ref[...] = jnp.dot(x_ref[...], w_ref[...],
                         preferred_element_type=jnp.float32)


def _combine_kernel(p_ref, b_ref, o_ref):
    o_ref[...] = p_ref[0] + p_ref[1] + b_ref[...]


def _lane_bcast(v16, j):
    idx = jnp.full((L, 1), j, jnp.int32)
    return lax.gather(
        v16, idx,
        lax.GatherDimensionNumbers(offset_dims=(), collapsed_slice_dims=(0,),
                                   start_index_map=(0,)),
        slice_sizes=(1,),
        mode=lax.GatherScatterMode.PROMISE_IN_BOUNDS)


def _make_sc_scatter(n, d, n_sup, sup, B):
    mesh = plsc.VectorSubcoreMesh(core_axis_name="c", subcore_axis_name="s")
    rc = B
    nrc = pl.cdiv(n, rc)
    rc_per_tile = pl.cdiv(nrc, NS)
    groups = B // L

    assert sup % 2 == 1

    @functools.partial(
        pl.kernel,
        out_type=jax.ShapeDtypeStruct((NC, n, d), jnp.float32),
        mesh=mesh,
        scratch_types=[
            pltpu.VMEM((sup, B), jnp.int32),
            pltpu.VMEM((sup, B), jnp.int32),
            pltpu.VMEM((sup, B), jnp.float32),
            pltpu.VMEM((B, d), jnp.float32),
            pltpu.VMEM((B, d), jnp.float32),
            pltpu.VMEM_SHARED((n, d), jnp.float32),
            pltpu.SemaphoreType.DMA,
            pltpu.SemaphoreType.DMA,
            pltpu.SemaphoreType.DMA,
            pltpu.SemaphoreType.DMA,
            pltpu.SemaphoreType.DMA,
            pltpu.SemaphoreType.DMA,
        ],
    )
    def sc_scatter(ei_hbm, adj_hbm, hid_hbm, out_hbm,
                   dst_v, src_v, adj_v, rows0, rows1, acc,
                   g0a, g0b, g1a, g1b, s0, s1):
        c = lax.axis_index("c")
        s = lax.axis_index("s")
        wid = s * NC + c

        def _scale(rows, k):
            def _group(g, carry2):
                a16 = adj_v[k, pl.ds(g * L, L)]
                for jj in range(L):
                    av = _lane_bcast(a16, jj)
                    i = g * L + jj
                    for jd in range(d // L):
                        rows[i, pl.ds(jd * L, L)] = (
                            rows[i, pl.ds(jd * L, L)] * av)
                return carry2
            lax.fori_loop(0, groups, _group, 0)

        def _zrow(i, carry):
            for j in range(d // L):
                rows0[i, pl.ds(j * L, L)] = jnp.zeros((L,), jnp.float32)
            return carry
        lax.fori_loop(0, rc, _zrow, 0)
        for k in range(rc_per_tile):
            cid = k * NS + s

            @pl.when(cid < nrc)
            def _():
                off = pl.multiple_of(cid * rc, 8)
                pltpu.sync_copy(rows0, acc.at[pl.ds(off, rc)])
        plsc.subcore_barrier()

        h = B // 2

        def _g2(k, rows, sa, sb):
            pltpu.async_copy(hid_hbm.at[src_v.at[k, pl.ds(0, h)]],
                             rows.at[pl.ds(0, h)], sa)
            pltpu.async_copy(hid_hbm.at[src_v.at[k, pl.ds(h, h)]],
                             rows.at[pl.ds(h, h)], sb)

        def _w2(k, rows, sa, sb):
            pltpu.make_async_copy(hid_hbm.at[src_v.at[k, pl.ds(0, h)]],
                                  rows.at[pl.ds(0, h)], sa).wait()
            pltpu.make_async_copy(hid_hbm.at[src_v.at[k, pl.ds(h, h)]],
                                  rows.at[pl.ds(h, h)], sb).wait()

        def _sup(m, carry):
            pltpu.sync_copy(ei_hbm.at[0, wid, m], dst_v)
            pltpu.sync_copy(ei_hbm.at[1, wid, m], src_v)
            pltpu.sync_copy(adj_hbm.at[wid, m], adj_v)
            _g2(0, rows0, g0a, g0b)

            def _pair(j, carry1):
                k0 = 2 * j
                k1 = 2 * j + 1
                _w2(k0, rows0, g0a, g0b)
                _g2(k1, rows1, g1a, g1b)
                _scale(rows0, k0)
                cp_s0 = pltpu.async_copy(rows0, acc.at[dst_v.at[k0]], s0,
                                         add=True)
                _w2(k1, rows1, g1a, g1b)
                cp_s0.wait()
                _g2(k1 + 1, rows0, g0a, g0b)
                _scale(rows1, k1)
                pltpu.async_copy(rows1, acc.at[dst_v.at[k1]], s1,
                                 add=True).wait()
                return carry1
            lax.fori_loop(0, sup // 2, _pair, 0)

            kt = sup - 1
            _w2(kt, rows0, g0a, g0b)
            _scale(rows0, kt)
            pltpu.sync_copy(rows0, acc.at[dst_v.at[kt]], add=True)
            return carry
        lax.fori_loop(0, n_sup, _sup, 0)
        plsc.subcore_barrier()

        for k in range(rc_per_tile):
            cid = k * NS + s

            @pl.when(cid < nrc)
            def _():
                off = pl.multiple_of(cid * rc, 8)
                pltpu.sync_copy(acc.at[pl.ds(off, rc)],
                                out_hbm.at[c, pl.ds(off, rc)])

    return sc_scatter


def kernel(x, edge_index, adj_values, W, b):
    n, d_in = x.shape
    d = W.shape[1]
    e = edge_index.shape[1]

    epw = e // NW
    B = 80
    sup = 25
    n_sup = epw // (B * sup)

    hidden = pl.pallas_call(
        _matmul_kernel,
        out_shape=jax.ShapeDtypeStruct((n, d), jnp.float32),
    )(x, W)

    ei_r = edge_index.reshape(2, NW, n_sup, sup, B)
    adj_r = adj_values.reshape(NW, n_sup, sup, B)

    psum = _make_sc_scatter(n, d, n_sup, sup, B)(
        ei_r, adj_r, hidden)

    out = pl.pallas_call(
        _combine_kernel,
        out_shape=jax.ShapeDtypeStruct((n, d), jnp.float32),
    )(psum, b.reshape(1, d))
    return out
